# baseline (speedup 1.0000x reference)
import functools

import jax
import jax.numpy as jnp
from jax import lax
from jax.experimental import pallas as pl
from jax.experimental.pallas import tpu as pltpu
from jax.experimental.pallas import tpu_sc as plsc

_D = 32
_NC = 2
_NS = 16
_NW = _NC * _NS

_HIST = 50
_BATCH = 4096
_VOCAB = 1000000

_CHUNK = 256
_CPH = _BATCH // _CHUNK
_NU = _HIST * _CPH
_STEPS = _NU // _NW

_NSLAB = _VOCAB // 128
_SROWS = _VOCAB // 4
_BS = 1
_NB = _NSLAB // _BS

_MAGIC = 305419896


def _make_phys_kernel():
    mesh = plsc.VectorSubcoreMesh(core_axis_name="c", subcore_axis_name="s")

    @functools.partial(
        pl.kernel,
        mesh=mesh,
        out_type=(
            jax.ShapeDtypeStruct((_HIST, _D, _BATCH), jnp.float32),
            jax.ShapeDtypeStruct((_SROWS, 128), jnp.float32),
            jax.ShapeDtypeStruct((16, 128), jnp.int32),
        ),
        scratch_types=[
            [pltpu.VMEM((_D, 128 * _BS), jnp.float32) for _ in range(2)],
            [pltpu.VMEM((32 * _BS, 128), jnp.float32) for _ in range(2)],
            pltpu.VMEM((_D, 128), jnp.float32),
            [pltpu.VMEM((8, _CHUNK), jnp.int32) for _ in range(2)],
            [pltpu.VMEM((2, 128), jnp.int32) for _ in range(2)],
            [pltpu.VMEM((_CHUNK,), jnp.int32) for _ in range(2)],
            [pltpu.VMEM((_CHUNK, 128), jnp.float32) for _ in range(2)],
            [[pltpu.VMEM((_D, 128), jnp.float32) for _ in range(2)]
             for _ in range(2)],
            pltpu.VMEM((8, 128), jnp.int32),
            pltpu.VMEM((8, 128), jnp.int32),
            [pltpu.SemaphoreType.DMA for _ in range(2)],
            [pltpu.SemaphoreType.DMA for _ in range(2)],
            [pltpu.SemaphoreType.DMA for _ in range(2)],
            [pltpu.SemaphoreType.DMA for _ in range(2)],
            pltpu.SemaphoreType.DMA,
        ],
        compiler_params=pltpu.CompilerParams(
            use_tc_tiling_on_sc=True, needs_layout_passes=False
        ),
    )
    def phys_kernel(idxT, tabT, tailS, outP, s_hbm, flag,
                    tbufs, sbufs, skew, idxbufs, glists, mrows, gbufs, obufs,
                    magic_v, fbuf, isems, osems, gsems, wsems, dsem):
        cid = lax.axis_index("c")
        sid = lax.axis_index("s")
        wid = sid * _NC + cid
        iota = lax.iota(jnp.int32, 16)

        b_start = (wid * _NB) // _NW
        b_end = ((wid + 1) * _NB) // _NW

        def in_slice(b):
            off = pl.multiple_of(b * (128 * _BS), 128)
            return tabT.at[:, pl.ds(off, 128 * _BS)]

        def out_slice(b):
            off = pl.multiple_of(b * (32 * _BS), 8)
            return s_hbm.at[pl.ds(off, 32 * _BS), :]

        def fire_in(b, p):
            pltpu.async_copy(in_slice(b), tbufs[p], isems[p])

        def transpose_batch(p):
            tb, sb = tbufs[p], sbufs[p]
            for q in range(_BS):
                def skew_body(jg, _):
                    loads = []
                    for jj in range(8):
                        j = 8 * jg + jj
                        jrow = jnp.full((16,), j, jnp.int32)
                        rot = (iota + j) & 15
                        for kk in range(8):
                            v = tb[j, pl.ds(128 * q + 16 * kk, 16)]
                            loads.append((jrow, 16 * kk + rot, v))
                    for jrow, lanes, v in loads:
                        plsc.store_scatter(skew, [jrow, lanes], v)
                    return 0

                lax.fori_loop(0, _D // 8, skew_body, 0)

                def col_body(sg, _):
                    gots = []
                    for ss in range(4):
                        s = 4 * sg + ss
                        cbase = (4 * s) & ~15
                        for a in range(4):
                            c = 4 * s + a
                            lane = cbase + ((c + iota) & 15)
                            for j0 in (0, 16):
                                v = plsc.load_gather(skew, [iota + j0, lane])
                                gots.append((s, a, j0, v))
                    for s, a, j0, v in gots:
                        sb[32 * q + s, pl.ds(a * 32 + j0, 16)] = v
                    return 0

                lax.fori_loop(0, 8, col_body, 0)

        def stage1():
            nloc = b_end - b_start

            @pl.when(nloc > 0)
            def _():
                fire_in(b_start, 0)

            def group_body(g, _):
                for p in range(2):
                    b = b_start + 2 * g + p

                    @pl.when(b < b_end)
                    def _():
                        nxt = b + 1
                        pltpu.make_async_copy(
                            in_slice(b), tbufs[p], isems[p]
                        ).wait()

                        @pl.when(nxt < b_end)
                        def _():
                            fire_in(nxt, 1 - p)

                        @pl.when(b - 2 >= b_start)
                        def _():
                            pltpu.make_async_copy(
                                sbufs[p], out_slice(b - 2), osems[p]
                            ).wait()

                        transpose_batch(p)
                        pltpu.async_copy(sbufs[p], out_slice(b), osems[p])

                return 0

            lax.fori_loop(0, (nloc + 1) // 2, group_body, 0)
            for p in range(2):
                @pl.when(nloc >= p + 1)
                def _():
                    pltpu.make_async_copy(
                        sbufs[p], out_slice(b_start), osems[p]
                    ).wait()

        stage1()

        @pl.when(wid == _NW - 1)
        def _():
            pltpu.sync_copy(tailS, sbufs[0].at[pl.ds(0, 16), :])
            pltpu.sync_copy(
                sbufs[0].at[pl.ds(0, 16), :],
                s_hbm.at[pl.ds(_NSLAB * 32, 16), :],
            )

        magic_v[0, pl.ds(0, 16)] = jnp.full((16,), _MAGIC, jnp.int32)
        plsc.subcore_barrier()

        @pl.when(sid == 0)
        def _():
            pltpu.sync_copy(
                magic_v, flag.at[pl.ds(pl.multiple_of(8 * cid, 8), 8), :]
            )

        def poll_body(_):
            pltpu.sync_copy(
                flag.at[pl.ds(pl.multiple_of(8 * (1 - cid), 8), 8), :], fbuf
            )
            got = fbuf[0, pl.ds(0, 16)]
            ok = jax.lax.reduce_and(got == _MAGIC, axes=(0,))
            return jnp.logical_not(ok)

        lax.while_loop(lambda nd: nd, poll_body, jnp.bool_(True))

        def unit_of(step):
            u = step * _NW + wid
            h = u // _CPH
            c0 = pl.multiple_of((u % _CPH) * _CHUNK, 128)
            return h, c0

        def prep_and_fire(step, p):
            h, c0 = unit_of(step)
            pltpu.sync_copy(
                idxT.at[
                    pl.ds(pl.multiple_of((h // 8) * 8, 8), 8), pl.ds(c0, _CHUNK)
                ],
                idxbufs[p],
            )
            hr = h % 8
            for k in range(_CHUNK // 16):
                iv = idxbufs[p][hr, pl.ds(16 * k, 16)]
                glists[p][k // 8, pl.ds(16 * (k % 8), 16)] = (
                    jax.lax.shift_right_logical(iv, 2)
                )
                mrows[p][pl.ds(16 * k, 16)] = jax.lax.shift_left(iv & 3, 5)
            for q in range(_CHUNK // 128):
                pltpu.async_copy(
                    s_hbm.at[glists[p].at[q]],
                    gbufs[p].at[pl.ds(128 * q, 128), :],
                    gsems[p],
                )

        def extract(step, p):
            gb, mr = gbufs[p], mrows[p]
            for half in range(2):
                ob = obufs[p][half]

                def c_body(c, _):
                    lbase = 16 * c

                    def d_body(dg, _):
                        outs = []
                        for dd in range(4):
                            d = 4 * dg + dd
                            lane = lbase + ((iota + d) & 15)
                            b_l = 128 * half + lane
                            mvp = plsc.load_gather(mr, [b_l])
                            for j0 in (0, 16):
                                v = plsc.load_gather(
                                    gb, [b_l, mvp + (iota + j0)]
                                )
                                outs.append((j0, lane, v))
                        for j0, lane, v in outs:
                            plsc.store_scatter(ob, [iota + j0, lane], v)
                        return 0

                    lax.fori_loop(0, 4, d_body, 0)
                    return 0

                lax.fori_loop(0, 8, c_body, 0)

        def write_out(step, p):
            h, c0 = unit_of(step)
            for half in range(2):
                pltpu.async_copy(
                    obufs[p][half],
                    outP.at[h, :, pl.ds(c0 + 128 * half, 128)],
                    wsems[p],
                )

        def wait_write(step, p):
            h, c0 = unit_of(step)
            for half in range(2):
                pltpu.make_async_copy(
                    obufs[p][half],
                    outP.at[h, :, pl.ds(c0 + 128 * half, 128)],
                    wsems[p],
                ).wait()

        def drain_gather(p):
            pltpu.make_async_copy(
                s_hbm.at[pl.ds(0, _CHUNK)], gbufs[p], gsems[p]
            ).wait()

        prep_and_fire(0, 0)

        def s2_group(g, _):
            for p in range(2):
                step = 2 * g + p

                @pl.when(step < _STEPS)
                def _():
                    drain_gather(p)

                    @pl.when(step + 1 < _STEPS)
                    def _():
                        prep_and_fire(step + 1, 1 - p)

                    @pl.when(step - 2 >= 0)
                    def _():
                        wait_write(step - 2, p)

                    extract(step, p)
                    write_out(step, p)

            return 0

        lax.fori_loop(0, (_STEPS + 1) // 2, s2_group, 0)
        for step in (_STEPS - 2, _STEPS - 1):
            wait_write(step, step % 2)

    return phys_kernel


def kernel(indices, table):
    idxT = indices.T.astype(jnp.int32)
    tabT = table.T
    tailS = table[_NSLAB * 128:].reshape(16, 128)
    outP, _, _ = _make_phys_kernel()(idxT, tabT, tailS)
    return outP.transpose(2, 0, 1)
